# Initial kernel scaffold; baseline (speedup 1.0000x reference)
import jax
import jax.numpy as jnp
from jax.experimental import pallas as pl


def kernel(x, edge_index):
    raise NotImplementedError("write your pallas kernel here")


# speedup vs baseline: 11.6307x; 11.6307x over previous
import jax
import jax.numpy as jnp
from jax import lax
from jax.experimental import pallas as pl
from jax.experimental.pallas import tpu as pltpu
from jax.experimental.pallas import tpu_sc as plsc

N = 10000
D = 128
DH = 64
E = 320000
K = 4
NC = 2
NS = 16
L = 16

NP = 10240
NPT = NP // NS
IB = 16
EB = 10
ECH = EB * IB
EPAD = NS * ECH * 128 - E
RCH = 128
NB = 2 * (K + 1)
HR = 16


def _fori(n, body):
    lax.fori_loop(jnp.int32(0), jnp.int32(n), body, None)


def _rsqrt16(d):
    i = plsc.bitcast(d, jnp.int32)
    i = jnp.int32(0x5F3759DF) - (i >> 1)
    y = plsc.bitcast(i, jnp.float32)
    half = jnp.float32(0.5) * d
    for _ in range(3):
        y = y * (jnp.float32(1.5) - half * y * y)
    return y


def _mgcn_body(x_hbm, ridx_hbm, cidx_hbm, out_hbm,
               ridx_b, cidx_b, hist_v, work_v, gbuf_v, deg_v, dinv_v,
               g_sp, s_sp):
    c = lax.axis_index("c")
    s = lax.axis_index("s")
    nbase = s * jnp.int32(NPT)
    zeros16 = jnp.zeros((L,), jnp.float32)
    ones16 = jnp.ones((L,), jnp.float32)

    def zero_hist(z, _):
        for q in range(DH // L):
            hist_v[z, pl.ds(q * L, L)] = zeros16
        return _
    _fori(NS * HR, zero_hist)

    def hist_block(b, _):
        pltpu.sync_copy(ridx_hbm.at[s, b], ridx_b)

        def hist_chunk(j, _h):
            for i in range(128 // L):
                idx = ridx_b[j, pl.ds(i * L, L)]
                rr = idx // jnp.int32(NPT)
                flat = rr * jnp.int32(HR * DH) + (idx - rr * jnp.int32(NPT))
                plsc.addupdate_scatter(
                    hist_v, [flat >> jnp.int32(6), flat & jnp.int32(63)],
                    ones16)
            return _h
        _fori(IB, hist_chunk)
        return _
    _fori(EB, hist_block)
    pltpu.sync_copy(hist_v, s_sp.at[pl.ds(s * jnp.int32(NS * HR), NS * HR)])
    plsc.subcore_barrier()

    pltpu.sync_copy(s_sp.at[pl.ds(s * jnp.int32(HR), HR)], deg_v)
    for t in range(1, NS):
        pltpu.sync_copy(
            s_sp.at[pl.ds(jnp.int32(t * NS * HR) + s * jnp.int32(HR), HR)],
            work_v.at[pl.ds(0, HR)])

        def acc(i, _):
            r = i >> jnp.int32(2)
            sl = pl.ds((i & jnp.int32(3)) * jnp.int32(L), L)
            deg_v[r, sl] = deg_v[r, sl] + work_v[r, sl]
            return _
        _fori(40, acc)

    def mk_dinv(i, _):
        r = i >> jnp.int32(2)
        q = (i & jnp.int32(3)) * jnp.int32(L)
        dinv_v[pl.ds(r * jnp.int32(DH) + q, L)] = _rsqrt16(
            deg_v[r, pl.ds(q, L)] + jnp.float32(1.0))
        return _
    _fori(40, mk_dinv)
    plsc.subcore_barrier()

    for cc in range(NPT // RCH):
        rb = nbase + jnp.int32(cc * RCH)
        pltpu.sync_copy(x_hbm.at[c, pl.ds(rb, RCH)], work_v)
        pltpu.sync_copy(work_v, out_hbm.at[c, pl.ds(rb, RCH)])

        def scale_g0(gidx, _):
            dv = dinv_v[pl.ds(cc * RCH + gidx * L, L)]
            for i in range(L):
                n = gidx * L + i
                d1 = dv[i]
                for q in range(DH // L):
                    sl = pl.ds(q * L, L)
                    work_v[n, sl] = d1 * work_v[n, sl]
            return _
        _fori(RCH // L, scale_g0)
        pltpu.sync_copy(work_v, g_sp.at[pl.ds(rb, RCH)])

    def zero_work(r, _):
        for q in range(DH // L):
            work_v[r, pl.ds(q * L, L)] = zeros16
        return _
    _fori(RCH, zero_work)
    for cc in range(NPT // RCH):
        pltpu.sync_copy(work_v, s_sp.at[pl.ds(nbase + jnp.int32(cc * RCH), RCH)])
    plsc.subcore_barrier()

    def hop(k, _):
        def eblock(b, _e):
            pltpu.sync_copy(ridx_hbm.at[s, b], ridx_b)
            pltpu.sync_copy(cidx_hbm.at[s, b], cidx_b)

            def echunk(j, _a):
                pltpu.sync_copy(g_sp.at[ridx_b.at[j]], work_v)
                pltpu.sync_copy(work_v, s_sp.at[cidx_b.at[j]], add=True)
                return _a
            _fori(IB, echunk)
            return _e
        _fori(EB, eblock)
        plsc.subcore_barrier()

        blk = jnp.int32(2) * (k + jnp.int32(1)) + c
        for cc in range(NPT // RCH):
            rb = nbase + jnp.int32(cc * RCH)
            pltpu.sync_copy(s_sp.at[pl.ds(rb, RCH)], work_v)
            pltpu.sync_copy(g_sp.at[pl.ds(rb, RCH)], gbuf_v)

            def rescale(gidx, _b):
                dv = dinv_v[pl.ds(cc * RCH + gidx * L, L)]
                for i in range(L):
                    n = gidx * L + i
                    d1 = dv[i]
                    for q in range(DH // L):
                        sl = pl.ds(q * L, L)
                        h = d1 * (work_v[n, sl] + gbuf_v[n, sl])
                        work_v[n, sl] = h
                        gbuf_v[n, sl] = d1 * h
                return _b
            _fori(RCH // L, rescale)
            pltpu.sync_copy(work_v, out_hbm.at[blk, pl.ds(rb, RCH)])
            pltpu.sync_copy(gbuf_v, g_sp.at[pl.ds(rb, RCH)])

        _fori(RCH, zero_work)
        for cc in range(NPT // RCH):
            pltpu.sync_copy(work_v,
                            s_sp.at[pl.ds(nbase + jnp.int32(cc * RCH), RCH)])
        plsc.subcore_barrier()
        return _

    _fori(K, hop)


_mgcn = pl.kernel(
    _mgcn_body,
    out_type=jax.ShapeDtypeStruct((NB, NP, DH), jnp.float32),
    mesh=plsc.VectorSubcoreMesh(core_axis_name="c", subcore_axis_name="s",
                                num_cores=NC, num_subcores=NS),
    scratch_types=[
        pltpu.VMEM((IB, 128), jnp.int32),
        pltpu.VMEM((IB, 128), jnp.int32),
        pltpu.VMEM((NS * HR, DH), jnp.float32),
        pltpu.VMEM((RCH, DH), jnp.float32),
        pltpu.VMEM((RCH, DH), jnp.float32),
        pltpu.VMEM((HR, DH), jnp.float32),
        pltpu.VMEM((NPT,), jnp.float32),
        pltpu.VMEM_SHARED((NP, DH), jnp.float32),
        pltpu.VMEM_SHARED((NP, DH), jnp.float32),
    ],
    compiler_params=pltpu.CompilerParams(use_tc_tiling_on_sc=False,
                                         needs_layout_passes=False),
)


def kernel(x, edge_index):
    ei = edge_index.astype(jnp.int32)
    pad = jnp.full((2, EPAD), N, dtype=jnp.int32)
    eip = jnp.concatenate([ei, pad], axis=1)
    ridx = eip[0].reshape(NS, EB, IB, 128)
    cidx = eip[1].reshape(NS, EB, IB, 128)
    xr = jnp.pad(x.astype(jnp.float32).reshape(N, NC, DH).transpose(1, 0, 2),
                 ((0, 0), (0, NP - N), (0, 0)))
    out = _mgcn(xr, ridx, cidx)
    return out[:, :N].transpose(1, 0, 2).reshape(N, (K + 1) * D)
